# baseline (device time: 20725 ns/iter reference)
import jax
import jax.numpy as jnp
from jax import lax
from jax.experimental import pallas as pl
from jax.experimental.pallas import tpu as pltpu

M = 1024
D = 512
BLK = 512
Q = 256


def kernel(partial, gamma):
    def body(x_ref, g_ref, out_ref, recv_ref, x_send_sem, x_recv_sem,
             y_send_sem, y_recv_sem):
        my_x = lax.axis_index("x")
        my_y = lax.axis_index("y")
        other_x = 1 - my_x
        other_y = 1 - my_y

        barrier_sem = pltpu.get_barrier_semaphore()
        pl.semaphore_signal(barrier_sem, inc=1, device_id=(other_x, my_y),
                            device_id_type=pl.DeviceIdType.MESH)
        pl.semaphore_signal(barrier_sem, inc=1, device_id=(my_x, other_y),
                            device_id_type=pl.DeviceIdType.MESH)
        pl.semaphore_wait(barrier_sem, 2)

        peer_q0 = other_x * BLK + my_y * Q
        my_q0 = my_x * BLK + my_y * Q
        rdma_x = pltpu.make_async_remote_copy(
            src_ref=x_ref.at[0, pl.ds(peer_q0, Q), :],
            dst_ref=recv_ref,
            send_sem=x_send_sem,
            recv_sem=x_recv_sem,
            device_id=(other_x, my_y),
            device_id_type=pl.DeviceIdType.MESH,
        )
        rdma_x.start()
        rdma_x.wait()

        ysum = x_ref[0, pl.ds(my_q0, Q), :] + recv_ref[:, :]
        mean_sq = jnp.sum(ysum * ysum, axis=-1, keepdims=True) * (1.0 / D)
        inv_rms = lax.rsqrt(mean_sq + 1e-6)
        out_q = ysum * inv_rms * jnp.reshape(g_ref[...], (1, D))
        out_ref[pl.ds(my_y * Q, Q), :] = out_q

        rdma_y = pltpu.make_async_remote_copy(
            src_ref=out_ref.at[pl.ds(my_y * Q, Q), :],
            dst_ref=out_ref.at[pl.ds(my_y * Q, Q), :],
            send_sem=y_send_sem,
            recv_sem=y_recv_sem,
            device_id=(my_x, other_y),
            device_id_type=pl.DeviceIdType.MESH,
        )
        rdma_y.start()
        rdma_y.wait()

    return pl.pallas_call(
        body,
        out_shape=jax.ShapeDtypeStruct((BLK, D), jnp.float32),
        in_specs=[
            pl.BlockSpec(memory_space=pltpu.VMEM),
            pl.BlockSpec(memory_space=pltpu.VMEM),
        ],
        out_specs=pl.BlockSpec(memory_space=pltpu.VMEM),
        scratch_shapes=[
            pltpu.VMEM((Q, D), jnp.float32),
            pltpu.SemaphoreType.DMA,
            pltpu.SemaphoreType.DMA,
            pltpu.SemaphoreType.DMA,
            pltpu.SemaphoreType.DMA,
        ],
        compiler_params=pltpu.CompilerParams(collective_id=0),
    )(partial, gamma)


# device time: 16521 ns/iter; 1.2545x vs baseline; 1.2545x over previous
import jax
import jax.numpy as jnp
from jax import lax
from jax.experimental import pallas as pl
from jax.experimental.pallas import tpu as pltpu

M = 1024
D = 512
BLK = 512
Q = 256
NC = 4
CR = Q // NC


def kernel(partial, gamma):
    def body(x_ref, g_ref, out_ref, recv_ref, x_send_sems, x_recv_sems,
             y_send_sems, y_recv_sems):
        my_x = lax.axis_index("x")
        my_y = lax.axis_index("y")
        other_x = 1 - my_x
        other_y = 1 - my_y

        barrier_sem = pltpu.get_barrier_semaphore()
        pl.semaphore_signal(barrier_sem, inc=1, device_id=(other_x, my_y),
                            device_id_type=pl.DeviceIdType.MESH)
        pl.semaphore_signal(barrier_sem, inc=1, device_id=(my_x, other_y),
                            device_id_type=pl.DeviceIdType.MESH)
        pl.semaphore_wait(barrier_sem, 2)

        peer_q0 = other_x * BLK + my_y * Q
        my_q0 = my_x * BLK + my_y * Q

        rdmas_x = []
        for c in range(NC):
            r = pltpu.make_async_remote_copy(
                src_ref=x_ref.at[0, pl.ds(peer_q0 + c * CR, CR), :],
                dst_ref=recv_ref.at[pl.ds(c * CR, CR), :],
                send_sem=x_send_sems.at[c],
                recv_sem=x_recv_sems.at[c],
                device_id=(other_x, my_y),
                device_id_type=pl.DeviceIdType.MESH,
            )
            r.start()
            rdmas_x.append(r)

        g_row = jnp.reshape(g_ref[...], (1, D))
        rdmas_y = []
        for c in range(NC):
            rdmas_x[c].wait_recv()
            ysum = (x_ref[0, pl.ds(my_q0 + c * CR, CR), :]
                    + recv_ref[pl.ds(c * CR, CR), :])
            mean_sq = jnp.sum(ysum * ysum, axis=-1, keepdims=True) * (1.0 / D)
            out_rows = ysum * lax.rsqrt(mean_sq + 1e-6) * g_row
            off = my_y * Q + c * CR
            out_ref[pl.ds(off, CR), :] = out_rows
            r = pltpu.make_async_remote_copy(
                src_ref=out_ref.at[pl.ds(off, CR), :],
                dst_ref=out_ref.at[pl.ds(off, CR), :],
                send_sem=y_send_sems.at[c],
                recv_sem=y_recv_sems.at[c],
                device_id=(my_x, other_y),
                device_id_type=pl.DeviceIdType.MESH,
            )
            r.start()
            rdmas_y.append(r)

        for c in range(NC):
            rdmas_y[c].wait_recv()
            rdmas_y[c].wait_send()
            rdmas_x[c].wait_send()

    return pl.pallas_call(
        body,
        out_shape=jax.ShapeDtypeStruct((BLK, D), jnp.float32),
        in_specs=[
            pl.BlockSpec(memory_space=pltpu.VMEM),
            pl.BlockSpec(memory_space=pltpu.VMEM),
        ],
        out_specs=pl.BlockSpec(memory_space=pltpu.VMEM),
        scratch_shapes=[
            pltpu.VMEM((Q, D), jnp.float32),
            pltpu.SemaphoreType.DMA((NC,)),
            pltpu.SemaphoreType.DMA((NC,)),
            pltpu.SemaphoreType.DMA((NC,)),
            pltpu.SemaphoreType.DMA((NC,)),
        ],
        compiler_params=pltpu.CompilerParams(collective_id=0),
    )(partial, gamma)
